# speedup vs baseline: 1.0121x; 1.0121x over previous
import jax
import jax.numpy as jnp
from jax.experimental import pallas as pl
from jax.experimental.pallas import tpu as pltpu

N = 10000
F = 128
RB = 400


def _pass1_body(adj_ref, x_ref, W1_ref, b1_ref, W2_ref, Wc_ref,
                u_ref, s1_ref, v_ref):
    i = pl.program_id(0)

    @pl.when(i == 0)
    def _init():
        s1_ref[...] = jnp.dot(x_ref[...], W1_ref[...],
                              preferred_element_type=jnp.float32)
        v_ref[...] = jnp.dot(W2_ref[...], Wc_ref[...],
                             preferred_element_type=jnp.float32)

    z = jnp.dot(adj_ref[...], s1_ref[...],
                preferred_element_type=jnp.float32)
    z = jnp.maximum(z + b1_ref[...], 0.0)
    u_ref[...] = jnp.dot(z, v_ref[...], preferred_element_type=jnp.float32)


def _pass2_body(adj_ref, u_ref, c_ref, out_ref):
    out_ref[...] = jnp.dot(adj_ref[...], u_ref[...],
                           preferred_element_type=jnp.float32) + c_ref[0, 0]


def kernel(adj, x, W1, b1, W2, b2, Wc, bc, We1, be1, We2, be2, Wfc, bfc):
    del We1, be1, We2, be2, Wfc, bfc
    b1_2d = b1.reshape(1, F)

    grid = (N // RB,)
    u = pl.pallas_call(
        _pass1_body,
        grid=grid,
        in_specs=[
            pl.BlockSpec((RB, N), lambda i: (i, 0)),
            pl.BlockSpec((N, F), lambda i: (0, 0)),
            pl.BlockSpec((F, F), lambda i: (0, 0)),
            pl.BlockSpec((1, F), lambda i: (0, 0)),
            pl.BlockSpec((F, F), lambda i: (0, 0)),
            pl.BlockSpec((F, 1), lambda i: (0, 0)),
        ],
        out_specs=pl.BlockSpec((RB, 1), lambda i: (i, 0)),
        out_shape=jax.ShapeDtypeStruct((N, 1), jnp.float32),
        scratch_shapes=[
            pltpu.VMEM((N, F), jnp.float32),
            pltpu.VMEM((F, 1), jnp.float32),
        ],
    )(adj, x, W1, b1_2d, W2, Wc)

    c = (b2.reshape(1, F) @ Wc + bc).reshape(1, 1)

    label = pl.pallas_call(
        _pass2_body,
        grid=grid,
        in_specs=[
            pl.BlockSpec((RB, N), lambda i: (i, 0)),
            pl.BlockSpec((N, 1), lambda i: (0, 0)),
            pl.BlockSpec((1, 1), lambda i: (0, 0), memory_space=pltpu.SMEM),
        ],
        out_specs=pl.BlockSpec((RB, 1), lambda i: (i, 0)),
        out_shape=jax.ShapeDtypeStruct((N, 1), jnp.float32),
    )(adj, u, c)
    return label


# speedup vs baseline: 1.0517x; 1.0391x over previous
import jax
import jax.numpy as jnp
from jax.experimental import pallas as pl
from jax.experimental.pallas import tpu as pltpu

N = 10000
F = 128
RB = 400
NBLK = N // RB


def _body(adj_ref, x_ref, W1_ref, b1_ref, W2_ref, b2_ref, Wc_ref, bc_ref,
          out_ref, u_ref, s1_ref, v_ref):
    i = pl.program_id(0)

    @pl.when(i == 0)
    def _init():
        s1_ref[...] = jnp.dot(x_ref[...], W1_ref[...],
                              preferred_element_type=jnp.float32)
        v_ref[...] = jnp.dot(W2_ref[...], Wc_ref[...],
                             preferred_element_type=jnp.float32)

    @pl.when(i < NBLK)
    def _phase1():
        z = jnp.dot(adj_ref[...], s1_ref[...],
                    preferred_element_type=jnp.float32)
        z = jnp.maximum(z + b1_ref[...], 0.0)
        blk = i * RB
        u_ref[pl.ds(blk, RB), :] = jnp.dot(z, v_ref[...],
                                           preferred_element_type=jnp.float32)

    @pl.when(i >= NBLK)
    def _phase2():
        c = jnp.dot(b2_ref[...], Wc_ref[...],
                    preferred_element_type=jnp.float32) + bc_ref[...]
        out_ref[...] = jnp.dot(adj_ref[...], u_ref[...],
                               preferred_element_type=jnp.float32) + c


def kernel(adj, x, W1, b1, W2, b2, Wc, bc, We1, be1, We2, be2, Wfc, bfc):
    del We1, be1, We2, be2, Wfc, bfc
    b1_2d = b1.reshape(1, F)
    b2_2d = b2.reshape(1, F)
    bc_2d = bc.reshape(1, 1)

    label = pl.pallas_call(
        _body,
        grid=(2 * NBLK,),
        in_specs=[
            pl.BlockSpec((RB, N), lambda i: (i % NBLK, 0)),
            pl.BlockSpec((N, F), lambda i: (0, 0)),
            pl.BlockSpec((F, F), lambda i: (0, 0)),
            pl.BlockSpec((1, F), lambda i: (0, 0)),
            pl.BlockSpec((F, F), lambda i: (0, 0)),
            pl.BlockSpec((1, F), lambda i: (0, 0)),
            pl.BlockSpec((F, 1), lambda i: (0, 0)),
            pl.BlockSpec((1, 1), lambda i: (0, 0)),
        ],
        out_specs=pl.BlockSpec(
            (RB, 1), lambda i: (jnp.where(i < NBLK, 0, i - NBLK), 0)),
        out_shape=jax.ShapeDtypeStruct((N, 1), jnp.float32),
        scratch_shapes=[
            pltpu.VMEM((N, 1), jnp.float32),
            pltpu.VMEM((N, F), jnp.float32),
            pltpu.VMEM((F, 1), jnp.float32),
        ],
    )(adj, x, W1, b1_2d, W2, b2_2d, Wc, bc_2d)
    return label


# speedup vs baseline: 1.0563x; 1.0044x over previous
import jax
import jax.numpy as jnp
from jax.experimental import pallas as pl
from jax.experimental.pallas import tpu as pltpu

N = 10000
F = 128
RB = 200
NSTEP = N // (2 * RB)


def _body(adjA_ref, adjB_ref, x_ref, W1_ref, b1_ref, W2_ref, b2_ref,
          Wc_ref, bc_ref, out_ref, u_ref, s1_ref, v_ref):
    i = pl.program_id(0)

    @pl.when(i == 0)
    def _init():
        s1_ref[...] = jnp.dot(x_ref[...], W1_ref[...],
                              preferred_element_type=jnp.float32)
        v_ref[...] = jnp.dot(W2_ref[...], Wc_ref[...],
                             preferred_element_type=jnp.float32)

    @pl.when(i < NSTEP)
    def _phase1():
        zA = jnp.dot(adjA_ref[...], s1_ref[...],
                     preferred_element_type=jnp.float32)
        zB = jnp.dot(adjB_ref[...], s1_ref[...],
                     preferred_element_type=jnp.float32)
        zA = jnp.maximum(zA + b1_ref[...], 0.0)
        zB = jnp.maximum(zB + b1_ref[...], 0.0)
        blk = i * 2 * RB
        u_ref[pl.ds(blk, RB), :] = jnp.dot(
            zA, v_ref[...], preferred_element_type=jnp.float32)
        u_ref[pl.ds(blk + RB, RB), :] = jnp.dot(
            zB, v_ref[...], preferred_element_type=jnp.float32)

    @pl.when(i >= NSTEP)
    def _phase2():
        c = jnp.dot(b2_ref[...], Wc_ref[...],
                    preferred_element_type=jnp.float32) + bc_ref[...]
        out_ref[:RB, :] = jnp.dot(adjA_ref[...], u_ref[...],
                                  preferred_element_type=jnp.float32) + c
        out_ref[RB:, :] = jnp.dot(adjB_ref[...], u_ref[...],
                                  preferred_element_type=jnp.float32) + c


def kernel(adj, x, W1, b1, W2, b2, Wc, bc, We1, be1, We2, be2, Wfc, bfc):
    del We1, be1, We2, be2, Wfc, bfc
    b1_2d = b1.reshape(1, F)
    b2_2d = b2.reshape(1, F)
    bc_2d = bc.reshape(1, 1)

    label = pl.pallas_call(
        _body,
        grid=(2 * NSTEP,),
        in_specs=[
            pl.BlockSpec((RB, N), lambda i: (2 * (i % NSTEP), 0)),
            pl.BlockSpec((RB, N), lambda i: (2 * (i % NSTEP) + 1, 0)),
            pl.BlockSpec((N, F), lambda i: (0, 0)),
            pl.BlockSpec((F, F), lambda i: (0, 0)),
            pl.BlockSpec((1, F), lambda i: (0, 0)),
            pl.BlockSpec((F, F), lambda i: (0, 0)),
            pl.BlockSpec((1, F), lambda i: (0, 0)),
            pl.BlockSpec((F, 1), lambda i: (0, 0)),
            pl.BlockSpec((1, 1), lambda i: (0, 0)),
        ],
        out_specs=pl.BlockSpec(
            (2 * RB, 1), lambda i: (jnp.where(i < NSTEP, 0, i - NSTEP), 0)),
        out_shape=jax.ShapeDtypeStruct((N, 1), jnp.float32),
        scratch_shapes=[
            pltpu.VMEM((N, 1), jnp.float32),
            pltpu.VMEM((N, F), jnp.float32),
            pltpu.VMEM((F, 1), jnp.float32),
        ],
    )(adj, adj, x, W1, b1_2d, W2, b2_2d, Wc, bc_2d)
    return label
